# Initial kernel scaffold; baseline (speedup 1.0000x reference)
import jax
import jax.numpy as jnp
from jax.experimental import pallas as pl


def kernel(x, edge_index, W1, b1, W2, b2):
    raise NotImplementedError("write your pallas kernel here")


# speedup vs baseline: 20.7282x; 20.7282x over previous
import functools

import jax
import jax.numpy as jnp
from jax import lax
from jax.experimental import pallas as pl
from jax.experimental.pallas import tpu as pltpu
from jax.experimental.pallas import tpu_sc as plsc

N = 10000
E = 320000
IN_C = 128
HID = 256
OUT_C = 128
H = IN_C // 2

NC = 2
NS = 16
NW = NC * NS
K = 112
NCH_DEG = 90
NCH = 180
EPAD = NW * NCH_DEG * K
NACC = 10112
ROWS_PT = NACC // NS
_OUT_CHUNKS = [K] * (ROWS_PT // K) + ([ROWS_PT % K] if ROWS_PT % K else [])

R = 1000
GRID = N // R


def _deg_body(dst_hbm, zeros_hbm, ones_hbm, out_hbm, dst_v, zbuf, ones_v, acc_sh):
    cid = lax.axis_index("c")
    sid = lax.axis_index("s")
    wid = cid * NS + sid
    pltpu.sync_copy(dst_hbm.at[wid], dst_v)
    pltpu.sync_copy(zeros_hbm, zbuf)
    pltpu.sync_copy(ones_hbm, ones_v)
    pltpu.sync_copy(zbuf, acc_sh.at[pl.ds(sid * ROWS_PT, ROWS_PT)])
    plsc.subcore_barrier()

    def body(j, carry):
        pltpu.sync_copy(ones_v, acc_sh.at[dst_v.at[j]], add=True)
        return carry

    lax.fori_loop(0, NCH_DEG, body, 0)
    plsc.subcore_barrier()
    pltpu.sync_copy(acc_sh.at[pl.ds(sid * ROWS_PT, ROWS_PT)], zbuf)
    pltpu.sync_copy(zbuf, out_hbm.at[cid, pl.ds(sid * ROWS_PT, ROWS_PT)])


def _agg_body(y_hbm, src_hbm, dst_hbm, zeros_hbm, out_hbm,
              src_v, dst_v, buf0, buf1, acc_sh, sem0, sem1):
    cid = lax.axis_index("c")
    sid = lax.axis_index("s")
    pltpu.sync_copy(src_hbm.at[cid, sid], src_v)
    pltpu.sync_copy(dst_hbm.at[sid], dst_v)
    pltpu.sync_copy(zeros_hbm, buf0)
    base = sid * ROWS_PT
    off = 0
    for sz in _OUT_CHUNKS:
        pltpu.sync_copy(buf0.at[pl.ds(0, sz)], acc_sh.at[pl.ds(base + off, sz)])
        off += sz
    plsc.subcore_barrier()

    pltpu.async_copy(y_hbm.at[src_v.at[0]], buf0, sem0)

    def body(i, carry):
        j0 = 2 * i
        j1 = j0 + 1
        pltpu.async_copy(y_hbm.at[src_v.at[j1]], buf1, sem1)
        pltpu.make_async_copy(y_hbm.at[src_v.at[j0]], buf0, sem0).wait()
        pltpu.sync_copy(buf0, acc_sh.at[dst_v.at[j0]], add=True)

        @pl.when(j0 + 2 < NCH)
        def _():
            pltpu.async_copy(y_hbm.at[src_v.at[j0 + 2]], buf0, sem0)

        pltpu.make_async_copy(y_hbm.at[src_v.at[j1]], buf1, sem1).wait()
        pltpu.sync_copy(buf1, acc_sh.at[dst_v.at[j1]], add=True)
        return carry

    lax.fori_loop(0, NCH // 2, body, 0)
    plsc.subcore_barrier()
    off = 0
    for sz in _OUT_CHUNKS:
        pltpu.sync_copy(acc_sh.at[pl.ds(base + off, sz)], buf0.at[pl.ds(0, sz)])
        pltpu.sync_copy(buf0.at[pl.ds(0, sz)], out_hbm.at[cid, pl.ds(base + off, sz)])
        off += sz


@functools.lru_cache(maxsize=None)
def _sc_calls():
    mesh = plsc.VectorSubcoreMesh(core_axis_name="c", subcore_axis_name="s",
                                  num_cores=NC, num_subcores=NS)
    params = pltpu.CompilerParams(use_tc_tiling_on_sc=False)
    deg_call = pl.kernel(
        _deg_body,
        out_type=jax.ShapeDtypeStruct((NC, NACC, 16), jnp.float32),
        mesh=mesh,
        scratch_types=[
            pltpu.VMEM((NCH_DEG, K), jnp.int32),
            pltpu.VMEM((ROWS_PT, 16), jnp.float32),
            pltpu.VMEM((K, 16), jnp.float32),
            pltpu.VMEM_SHARED((NACC, 16), jnp.float32),
        ],
        compiler_params=params,
    )
    agg_call = pl.kernel(
        _agg_body,
        out_type=jax.ShapeDtypeStruct((NC, NACC, H), jnp.float32),
        mesh=mesh,
        scratch_types=[
            pltpu.VMEM((NCH, K), jnp.int32),
            pltpu.VMEM((NCH, K), jnp.int32),
            pltpu.VMEM((K, H), jnp.float32),
            pltpu.VMEM((K, H), jnp.float32),
            pltpu.VMEM_SHARED((NACC, H), jnp.float32),
            pltpu.SemaphoreType.DMA,
            pltpu.SemaphoreType.DMA,
        ],
        compiler_params=params,
    )
    return deg_call, agg_call


def _dinv(dA_ref, dB_ref):
    return lax.rsqrt(dA_ref[:, 0:1] + dB_ref[:, 0:1] + 1.0)


def _prep_kernel(dA_ref, dB_ref, x_ref, y_ref):
    dinv = _dinv(dA_ref, dB_ref)
    y_ref[0, :, :] = x_ref[:, 0:H] * dinv
    y_ref[1, :, :] = x_ref[:, H:IN_C] * dinv


def _mm_kernel(z_ref, y_ref, dA_ref, dB_ref, W1_ref, b1_ref, W2_ref, y2_ref):
    dinv = _dinv(dA_ref, dB_ref)
    s0 = (z_ref[0, :, :] + y_ref[0, :, :]) * dinv
    s1 = (z_ref[1, :, :] + y_ref[1, :, :]) * dinv
    h = (jnp.dot(s0, W1_ref[0:H, :], preferred_element_type=jnp.float32)
         + jnp.dot(s1, W1_ref[H:IN_C, :], preferred_element_type=jnp.float32)
         + b1_ref[...])
    h = jnp.maximum(h, 0.0)
    y2_ref[0, :, :] = jnp.dot(h, W2_ref[:, 0:H],
                              preferred_element_type=jnp.float32) * dinv
    y2_ref[1, :, :] = jnp.dot(h, W2_ref[:, H:OUT_C],
                              preferred_element_type=jnp.float32) * dinv


def _fin_kernel(z_ref, y_ref, dA_ref, dB_ref, b2_ref, out_ref):
    dinv = _dinv(dA_ref, dB_ref)
    out_ref[:, 0:H] = (z_ref[0, :, :] + y_ref[0, :, :]) * dinv + b2_ref[:, 0:H]
    out_ref[:, H:OUT_C] = (z_ref[1, :, :] + y_ref[1, :, :]) * dinv + b2_ref[:, H:OUT_C]


def _row_spec(w):
    return pl.BlockSpec((R, w), lambda i: (i, 0))


def _half_spec():
    return pl.BlockSpec((NC, R, H), lambda i: (0, i, 0))


def _full_spec(h, w):
    return pl.BlockSpec((h, w), lambda i: (0, 0))


@functools.lru_cache(maxsize=None)
def _tc_calls():
    prep = pl.pallas_call(
        _prep_kernel,
        grid=(GRID,),
        in_specs=[_row_spec(16), _row_spec(16), _row_spec(IN_C)],
        out_specs=_half_spec(),
        out_shape=jax.ShapeDtypeStruct((NC, N, H), jnp.float32),
    )
    mm = pl.pallas_call(
        _mm_kernel,
        grid=(GRID,),
        in_specs=[_half_spec(), _half_spec(), _row_spec(16), _row_spec(16),
                  _full_spec(IN_C, HID), _full_spec(1, HID),
                  _full_spec(HID, OUT_C)],
        out_specs=_half_spec(),
        out_shape=jax.ShapeDtypeStruct((NC, N, H), jnp.float32),
    )
    fin = pl.pallas_call(
        _fin_kernel,
        grid=(GRID,),
        in_specs=[_half_spec(), _half_spec(), _row_spec(16), _row_spec(16),
                  _full_spec(1, OUT_C)],
        out_specs=_row_spec(OUT_C),
        out_shape=jax.ShapeDtypeStruct((N, OUT_C), jnp.float32),
    )
    return prep, mm, fin


def kernel(x, edge_index, W1, b1, W2, b2):
    deg_call, agg_call = _sc_calls()
    prep, mm, fin = _tc_calls()

    pad = EPAD - E
    src = jnp.concatenate([edge_index[0], jnp.zeros((pad,), jnp.int32)])
    dst = jnp.concatenate([edge_index[1], jnp.full((pad,), N, jnp.int32)])
    dst_deg = dst.reshape(NW, NCH_DEG, K)
    src_agg = jnp.stack([src, src + N]).reshape(NC, NS, NCH, K)
    dst_agg = dst.reshape(NS, NCH, K)

    ones16 = jnp.ones((K, 16), jnp.float32)
    zeros16 = jnp.zeros((ROWS_PT, 16), jnp.float32)
    zerosK = jnp.zeros((K, H), jnp.float32)

    degp = deg_call(dst_deg, zeros16, ones16)
    dA = degp[0, :N]
    dB = degp[1, :N]

    y1 = prep(dA, dB, x)
    z1 = agg_call(y1.reshape(NC * N, H), src_agg, dst_agg, zerosK)
    y2 = mm(z1[:, :N], y1, dA, dB,
            W1, b1.reshape(1, HID), W2)
    z2 = agg_call(y2.reshape(NC * N, H), src_agg, dst_agg, zerosK)
    return fin(z2[:, :N], y2, dA, dB, b2.reshape(1, OUT_C))
